# baseline (speedup 1.0000x reference)
import functools

import jax
import jax.numpy as jnp
from jax import lax
from jax.experimental import pallas as pl
from jax.experimental.pallas import tpu as pltpu
from jax.experimental.pallas import tpu_sc as plsc

_NC = 2
_NS = 16
_NW = _NC * _NS
_C = 128
_RING = 4


@functools.lru_cache(maxsize=None)
def _build(N, D, V):
    per_w = N // _NW
    n_chunks = per_w // _C
    n_super = n_chunks // _RING
    scale = jnp.float32(float(D) ** -0.5)
    mesh = plsc.VectorSubcoreMesh(core_axis_name="c", subcore_axis_name="s")

    @functools.partial(
        pl.kernel,
        mesh=mesh,
        compiler_params=pltpu.CompilerParams(use_tc_tiling_on_sc=False),
        out_type=jax.ShapeDtypeStruct((N, D), jnp.float32),
        scratch_types=(
            [pltpu.VMEM((per_w,), jnp.int32)]
            + [pltpu.VMEM((_C, D), jnp.float32) for _ in range(_RING)]
            + [pltpu.SemaphoreType.DMA for _ in range(2 * _RING)]
        ),
    )
    def body(x_hbm, table_hbm, out_hbm, idx_v, *rest):
        rows = rest[:_RING]
        gsems = rest[_RING:2 * _RING]
        ssems = rest[2 * _RING:]
        wid = lax.axis_index("s") * _NC + lax.axis_index("c")
        base = wid * per_w

        pltpu.sync_copy(x_hbm.at[pl.ds(base, per_w)], idx_v)

        def g_copy(chunk, slot):
            return pltpu.make_async_copy(
                table_hbm.at[idx_v.at[pl.ds(chunk * _C, _C)]],
                rows[slot], gsems[slot])

        def s_copy(chunk, slot):
            return pltpu.make_async_copy(
                rows[slot], out_hbm.at[pl.ds(base + chunk * _C, _C)],
                ssems[slot])

        def scale_buf(buf):
            def row(i, carry):
                for j in range(D // 16):
                    buf[i, pl.ds(j * 16, 16)] = buf[i, pl.ds(j * 16, 16)] * scale
                return carry
            lax.fori_loop(0, _C, row, 0)

        for slot in range(_RING):
            g_copy(slot, slot).start()

        def super_step(g, carry):
            c0 = g * _RING
            for slot in range(_RING):
                g_copy(c0 + slot, slot).wait()
                scale_buf(rows[slot])
                s_copy(c0 + slot, slot).start()

            @pl.when(g + 1 < n_super)
            def _():
                for slot in range(_RING):
                    s_copy(c0 + slot, slot).wait()
                    g_copy(c0 + _RING + slot, slot).start()

            return carry

        lax.fori_loop(0, n_super, super_step, 0)

        for slot in range(_RING):
            s_copy(n_chunks - _RING + slot, slot).wait()

    return body


def kernel(x, table):
    B, S = x.shape
    V, D = table.shape
    N = B * S
    xf = x.reshape(N).astype(jnp.int32)
    out = _build(N, D, V)(xf, table)
    return out.reshape(B, S, D)


# speedup vs baseline: 1.0415x; 1.0415x over previous
import functools

import jax
import jax.numpy as jnp
from jax import lax
from jax.experimental import pallas as pl
from jax.experimental.pallas import tpu as pltpu
from jax.experimental.pallas import tpu_sc as plsc

_NC = 2
_NS = 16
_NW = _NC * _NS
_C = 128
_RING = 4


@functools.lru_cache(maxsize=None)
def _build(N, D, V):
    per_w = N // _NW
    n_chunks = per_w // _C
    n_super = n_chunks // _RING
    scale = jnp.float32(float(D) ** -0.5)
    mesh = plsc.VectorSubcoreMesh(core_axis_name="c", subcore_axis_name="s")

    @functools.partial(
        pl.kernel,
        mesh=mesh,
        compiler_params=pltpu.CompilerParams(use_tc_tiling_on_sc=False),
        out_type=jax.ShapeDtypeStruct((N, D), jnp.float32),
        scratch_types=(
            [pltpu.VMEM((per_w,), jnp.int32)]
            + [pltpu.VMEM((_C, D), jnp.float32) for _ in range(_RING)]
            + [pltpu.SemaphoreType.DMA for _ in range(2 * _RING)]
        ),
    )
    def body(x_hbm, table_hbm, out_hbm, idx_v, *rest):
        rows = rest[:_RING]
        gsems = rest[_RING:2 * _RING]
        ssems = rest[2 * _RING:]
        wid = lax.axis_index("s") * _NC + lax.axis_index("c")
        base = wid * per_w

        pltpu.sync_copy(x_hbm.at[pl.ds(base, per_w)], idx_v)

        def g_copy(chunk, slot):
            return pltpu.make_async_copy(
                table_hbm.at[idx_v.at[pl.ds(chunk * _C, _C)]],
                rows[slot], gsems[slot])

        def s_copy(chunk, slot):
            return pltpu.make_async_copy(
                rows[slot], out_hbm.at[pl.ds(base + chunk * _C, _C)],
                ssems[slot])

        def scale_buf(buf):
            @plsc.parallel_loop(0, _C, unroll=8)
            def _(i):
                for j in range(D // 16):
                    buf[i, pl.ds(j * 16, 16)] = buf[i, pl.ds(j * 16, 16)] * scale

        for slot in range(_RING):
            g_copy(slot, slot).start()

        def super_step(g, carry):
            c0 = g * _RING
            for slot in range(_RING):
                g_copy(c0 + slot, slot).wait()
                scale_buf(rows[slot])
                s_copy(c0 + slot, slot).start()

            @pl.when(g + 1 < n_super)
            def _():
                for slot in range(_RING):
                    s_copy(c0 + slot, slot).wait()
                    g_copy(c0 + _RING + slot, slot).start()

            return carry

        lax.fori_loop(0, n_super, super_step, 0)

        for slot in range(_RING):
            s_copy(n_chunks - _RING + slot, slot).wait()

    return body


def kernel(x, table):
    B, S = x.shape
    V, D = table.shape
    N = B * S
    xf = x.reshape(N).astype(jnp.int32)
    out = _build(N, D, V)(xf, table)
    return out.reshape(B, S, D)
